# baseline (speedup 1.0000x reference)
import functools

import jax
import jax.numpy as jnp
from jax import lax
from jax.experimental import pallas as pl
from jax.experimental.pallas import tpu as pltpu
from jax.experimental.pallas import tpu_sc as plsc

_NC = 2
_NS = 16
_NW = _NC * _NS

_IW = 512
_JB = 8
_IH = _IW // 2


@functools.lru_cache(maxsize=None)
def _build(n: int, s: int, dim: int):
    assert n % _IW == 0 and n // _IW == _NW
    n_jb = s // _JB
    assert s % _JB == 0
    mesh = plsc.VectorSubcoreMesh(core_axis_name="c", subcore_axis_name="s")

    @functools.partial(
        pl.kernel,
        mesh=mesh,
        out_type=jax.ShapeDtypeStruct((dim, s, n), jnp.float32),
        scratch_types=[
            pltpu.VMEM((9 * dim,), jnp.float32),
            pltpu.VMEM((_JB, _IW), jnp.int32),
            [pltpu.VMEM((_JB, _IH), jnp.float32)] * (2 * dim),
            pltpu.SemaphoreType.DMA,
            pltpu.SemaphoreType.DMA,
        ],
        compiler_params=pltpu.CompilerParams(
            use_tc_tiling_on_sc=True, needs_layout_passes=False),
    )
    def lut_kernel(idxt_hbm, tabtf_hbm, out_hbm,
                   tab_v, ibuf, obufs, sem_in, sem_out):
        cid = lax.axis_index("c")
        sid = lax.axis_index("s")
        wid = sid * _NC + cid
        i0 = wid * _IW

        pltpu.sync_copy(tabtf_hbm, tab_v)

        def dst_of(d, jt, h):
            return out_hbm.at[d, pl.ds(jt * _JB, _JB),
                              pl.ds(i0 + h * _IH, _IH)]

        def jblock(jt, carry):
            pltpu.async_copy(
                idxt_hbm.at[pl.ds(jt * _JB, _JB), pl.ds(i0, _IW)],
                ibuf, sem_in).wait()

            for h in range(2):
                bufs = obufs[h * dim:(h + 1) * dim]
                for d in range(dim):
                    @pl.when(jt > 0)
                    def _():
                        pltpu.make_async_copy(
                            dst_of(d, jt, h), bufs[d], sem_out).wait()

                @plsc.parallel_loop(0, _IH // 16, 1, unroll=4)
                def _(k):
                    for jr in range(_JB):
                        pv = ibuf[jr, pl.ds(h * _IH + 16 * k, 16)]
                        for d in range(dim):
                            bufs[d][jr, pl.ds(16 * k, 16)] = (
                                plsc.load_gather(tab_v, [pv + 9 * d]))

                for d in range(dim):
                    pltpu.async_copy(bufs[d], dst_of(d, jt, h), sem_out)
            return carry

        lax.fori_loop(0, n_jb, jblock, 0)
        for h in range(2):
            for d in range(dim):
                pltpu.make_async_copy(
                    dst_of(d, n_jb - 1, h), obufs[h * dim + d],
                    sem_out).wait()

    return lut_kernel


def kernel(lengths, table):
    n, s = lengths.shape
    _, dim = table.shape
    idxt = lengths.T
    tabtf = table.T.reshape(9 * dim)
    out_t = _build(n, s, dim)(idxt, tabtf)
    return out_t.transpose(2, 1, 0)


# speedup vs baseline: 2.7176x; 2.7176x over previous
import functools

import jax
import jax.numpy as jnp
from jax import lax
from jax.experimental import pallas as pl
from jax.experimental.pallas import tpu as pltpu
from jax.experimental.pallas import tpu_sc as plsc

_NC = 2
_NS = 16
_NW = _NC * _NS

_IW = 512
_JB = 8
_IH = _IW // 2


@functools.lru_cache(maxsize=None)
def _build(n: int, s: int, dim: int):
    assert n % _IW == 0 and n // _IW == _NW
    n_jb = s // _JB
    assert s % _JB == 0
    mesh = plsc.VectorSubcoreMesh(core_axis_name="c", subcore_axis_name="s")

    @functools.partial(
        pl.kernel,
        mesh=mesh,
        out_type=jax.ShapeDtypeStruct((dim, s, n), jnp.float32),
        scratch_types=[
            pltpu.VMEM((9 * dim,), jnp.float32),
            pltpu.VMEM((_JB, _IW), jnp.int32),
            [pltpu.VMEM((_JB, _IH), jnp.float32)] * (2 * dim),
            pltpu.SemaphoreType.DMA,
            pltpu.SemaphoreType.DMA,
        ],
        compiler_params=pltpu.CompilerParams(
            use_tc_tiling_on_sc=True, needs_layout_passes=False),
    )
    def lut_kernel(idxt_hbm, tabtf_hbm, out_hbm,
                   tab_v, ibuf, obufs, sem_in, sem_out):
        cid = lax.axis_index("c")
        sid = lax.axis_index("s")
        wid = sid * _NC + cid
        i0 = wid * _IW

        pltpu.sync_copy(tabtf_hbm, tab_v)

        def dst_of(d, jt, h):
            return out_hbm.at[d, pl.ds(jt * _JB, _JB),
                              pl.ds(i0 + h * _IH, _IH)]

        def jblock(jt, carry):
            pltpu.async_copy(
                idxt_hbm.at[pl.ds(jt * _JB, _JB), pl.ds(i0, _IW)],
                ibuf, sem_in).wait()

            for h in range(2):
                bufs = obufs[h * dim:(h + 1) * dim]
                for d in range(dim):
                    @pl.when(jt > 0)
                    def _():
                        pltpu.make_async_copy(
                            dst_of(d, jt, h), bufs[d], sem_out).wait()

                @plsc.parallel_loop(0, _IH // 16, 1, unroll=1)
                def _(k):
                    for jr in range(_JB):
                        pv = ibuf[jr, pl.ds(h * _IH + 16 * k, 16)]
                        for d in range(dim):
                            bufs[d][jr, pl.ds(16 * k, 16)] = (
                                plsc.load_gather(tab_v, [pv + 9 * d]))

                for d in range(dim):
                    pltpu.async_copy(bufs[d], dst_of(d, jt, h), sem_out)
            return carry

        lax.fori_loop(0, n_jb, jblock, 0)
        for h in range(2):
            for d in range(dim):
                pltpu.make_async_copy(
                    dst_of(d, n_jb - 1, h), obufs[h * dim + d],
                    sem_out).wait()

    return lut_kernel


def kernel(lengths, table):
    n, s = lengths.shape
    _, dim = table.shape
    idxt = lengths.T
    tabtf = table.T.reshape(9 * dim)
    out_t = _build(n, s, dim)(idxt, tabtf)
    return out_t.transpose(2, 1, 0)
